# Initial kernel scaffold; baseline (speedup 1.0000x reference)
import jax
import jax.numpy as jnp
from jax.experimental import pallas as pl


def kernel(init_emb, W_bdd, W_loop, bias, edge_index, etype):
    raise NotImplementedError("write your pallas kernel here")


# speedup vs baseline: 2.5382x; 2.5382x over previous
import functools

import jax
import jax.numpy as jnp
from jax import lax
from jax.experimental import pallas as pl
from jax.experimental.pallas import tpu as pltpu
from jax.experimental.pallas import tpu_sc as plsc

_N = 10000
_E = 160000
_NR = 4
_NBK = 4
_BI = 75
_BO = 128
_EP = 163840
_NC = 2
_NS = 16
_ET = _EP // _NS
_NBATCH = _ET // 128
_RPT = 624
_AGG_ROWS = _N + 8


def _msg_body(x_ref, w_ref, o_ref):
    o_ref[0, 0] = jnp.dot(x_ref[...], w_ref[0, 0],
                          preferred_element_type=jnp.float32)


def _tc_messages(xpad, wpad):
    tn = 400
    grid = (_N // tn, _NBK, _NR)
    return pl.pallas_call(
        _msg_body,
        grid=grid,
        in_specs=[
            pl.BlockSpec((tn, 128), lambda t, c, r: (t, c)),
            pl.BlockSpec((1, 1, 128, 128), lambda t, c, r: (r, c, 0, 0)),
        ],
        out_specs=pl.BlockSpec((1, 1, tn, 128), lambda t, c, r: (c, r, t, 0)),
        out_shape=jax.ShapeDtypeStruct((_NBK, _NR, _N, 128), jnp.float32),
    )(xpad, wpad)


def _self_body(x_ref, w_ref, b_ref, o_ref):
    o_ref[0] = jnp.dot(x_ref[...], w_ref[...],
                       preferred_element_type=jnp.float32) + b_ref[0:1, :]


def _tc_self(init_emb, w_loop, bias8):
    tn = 400
    grid = (_NBK, _N // tn)
    return pl.pallas_call(
        _self_body,
        grid=grid,
        in_specs=[
            pl.BlockSpec((tn, 300), lambda c, t: (t, 0)),
            pl.BlockSpec((300, 128), lambda c, t: (0, c)),
            pl.BlockSpec((8, 128), lambda c, t: (0, c)),
        ],
        out_specs=pl.BlockSpec((1, tn, 128), lambda c, t: (c, t, 0)),
        out_shape=jax.ShapeDtypeStruct((_NBK, _N, 128), jnp.float32),
    )(init_emb, w_loop, bias8)


def _sc_body(ylin, s4, srcf, etf, dst2d, out4,
             cidx_v, tmp_v, dst_v, rows_v, agg_sh, sem):
    c2 = lax.axis_index("c")
    s = lax.axis_index("s")
    e0 = s * _ET
    pltpu.sync_copy(srcf.at[pl.ds(e0, _ET)], cidx_v)
    pltpu.sync_copy(etf.at[pl.ds(e0, _ET)], tmp_v)
    pltpu.sync_copy(dst2d.at[pl.ds(s * _NBATCH, _NBATCH)], dst_v)
    r0 = s * _RPT
    tail = _N - _NS * _RPT

    def cbody(i, carry):
        off = i * 16
        cidx_v[pl.ds(off, 16)] = (
            tmp_v[pl.ds(off, 16)] * _N
            + cidx_v[pl.ds(off, 16)]
            + (c2 * 2) * (_NR * _N))
        return carry
    lax.fori_loop(0, _ET // 16, cbody, 0)

    for l in range(2):
        c = c2 * 2 + l
        pltpu.sync_copy(s4.at[c, pl.ds(r0, _RPT)],
                        agg_sh.at[pl.ds(r0, _RPT)])

        @pl.when(s == _NS - 1)
        def _():
            pltpu.sync_copy(s4.at[c, pl.ds(_NS * _RPT, tail)],
                            agg_sh.at[pl.ds(_NS * _RPT, tail)])

        if l > 0:
            def abody(i, carry):
                off = i * 16
                cidx_v[pl.ds(off, 16)] = cidx_v[pl.ds(off, 16)] + _NR * _N
                return carry
            lax.fori_loop(0, _ET // 16, abody, 0)

        plsc.subcore_barrier()

        def bbody(j, carry):
            idx = cidx_v.at[pl.ds(j * 128, 128)]
            pltpu.async_copy(ylin.at[idx], rows_v, sem).wait()
            pltpu.sync_copy(rows_v, agg_sh.at[dst_v.at[j]], add=True)
            return carry
        lax.fori_loop(0, _NBATCH, bbody, 0)

        plsc.subcore_barrier()

        pltpu.sync_copy(agg_sh.at[pl.ds(r0, _RPT)],
                        out4.at[c, pl.ds(r0, _RPT)])

        @pl.when(s == _NS - 1)
        def _():
            pltpu.sync_copy(agg_sh.at[pl.ds(_NS * _RPT, tail)],
                            out4.at[c, pl.ds(_NS * _RPT, tail)])


def _sc_aggregate(ylin, s4, srcp, etp, dst2d):
    mesh = plsc.VectorSubcoreMesh(core_axis_name="c", subcore_axis_name="s")
    run = functools.partial(
        pl.kernel,
        out_type=jax.ShapeDtypeStruct((_NBK, _N, 128), jnp.float32),
        mesh=mesh,
        scratch_types=[
            pltpu.VMEM((_ET,), jnp.int32),
            pltpu.VMEM((_ET,), jnp.int32),
            pltpu.VMEM((_NBATCH, 128), jnp.int32),
            pltpu.VMEM((128, 128), jnp.float32),
            pltpu.VMEM_SHARED((_AGG_ROWS, 128), jnp.float32),
            pltpu.SemaphoreType.DMA,
        ],
    )(_sc_body)
    return run(ylin, s4, srcp, etp, dst2d)


def kernel(init_emb, W_bdd, W_loop, bias, edge_index, etype):
    f32 = jnp.float32
    xpad = (jnp.zeros((_N, _NBK, 128), f32)
            .at[:, :, :_BI].set(init_emb.reshape(_N, _NBK, _BI))
            .reshape(_N, _NBK * 128))
    wpad = jnp.zeros((_NR, _NBK, 128, _BO), f32).at[:, :, :_BI, :].set(W_bdd)
    bias8 = jnp.broadcast_to(bias.reshape(1, 512), (8, 512))

    src = edge_index[0]
    dst = edge_index[1]
    pad = _EP - _E
    srcp = jnp.concatenate([src, jnp.zeros((pad,), jnp.int32)])
    etp = jnp.concatenate([etype, jnp.zeros((pad,), jnp.int32)])
    dst2d = jnp.concatenate(
        [dst, jnp.full((pad,), _N, jnp.int32)]).reshape(_EP // 128, 128)

    y4 = _tc_messages(xpad, wpad)
    s4 = _tc_self(init_emb, W_loop, bias8)
    out4 = _sc_aggregate(y4.reshape(_NBK * _NR * _N, 128),
                         s4, srcp, etp, dst2d)
    return out4.transpose(1, 0, 2).reshape(_N, _NR * 128)


# speedup vs baseline: 2.6557x; 1.0463x over previous
import functools

import jax
import jax.numpy as jnp
from jax import lax
from jax.experimental import pallas as pl
from jax.experimental.pallas import tpu as pltpu
from jax.experimental.pallas import tpu_sc as plsc

_N = 10000
_E = 160000
_NR = 4
_NBK = 4
_BI = 75
_BO = 128
_EP = 163840
_NC = 2
_NS = 16
_ET = _EP // _NS
_BQ = 64
_NBQ = _ET // _BQ
_STRIP = 2048
_RPT = 624
_AGG_ROWS = _N + 8


def _msg_body(x_ref, w_ref, o_ref):
    o_ref[0, 0] = jnp.dot(x_ref[...], w_ref[0, 0],
                          preferred_element_type=jnp.float32)


def _tc_messages(xpad, wpad):
    tn = 400
    grid = (_N // tn, _NBK, _NR)
    return pl.pallas_call(
        _msg_body,
        grid=grid,
        in_specs=[
            pl.BlockSpec((tn, 128), lambda t, c, r: (t, c)),
            pl.BlockSpec((1, 1, 128, 128), lambda t, c, r: (r, c, 0, 0)),
        ],
        out_specs=pl.BlockSpec((1, 1, tn, 128), lambda t, c, r: (c, r, t, 0)),
        out_shape=jax.ShapeDtypeStruct((_NBK, _NR, _N, 128), jnp.float32),
    )(xpad, wpad)


def _self_body(x_ref, w_ref, b_ref, o_ref):
    o_ref[0] = jnp.dot(x_ref[...], w_ref[...],
                       preferred_element_type=jnp.float32) + b_ref[0:1, :]


def _tc_self(init_emb, w_loop, bias8):
    tn = 400
    grid = (_NBK, _N // tn)
    return pl.pallas_call(
        _self_body,
        grid=grid,
        in_specs=[
            pl.BlockSpec((tn, 300), lambda c, t: (t, 0)),
            pl.BlockSpec((300, 128), lambda c, t: (0, c)),
            pl.BlockSpec((8, 128), lambda c, t: (0, c)),
        ],
        out_specs=pl.BlockSpec((1, tn, 128), lambda c, t: (c, t, 0)),
        out_shape=jax.ShapeDtypeStruct((_NBK, _N, 128), jnp.float32),
    )(init_emb, w_loop, bias8)


def _sc_body(ylin, s4, srcf, etf, dst2d, out4,
             cidx_v, tmp_v, dst_v, rows_a, rows_b, agg_sh, sem_a, sem_b):
    c2 = lax.axis_index("c")
    s = lax.axis_index("s")
    e0 = s * _ET
    pltpu.sync_copy(srcf.at[pl.ds(e0, _ET)], cidx_v)
    pltpu.sync_copy(dst2d.at[pl.ds(s * _NBQ, _NBQ)], dst_v)
    r0 = s * _RPT
    tail = _N - _NS * _RPT

    def sbody(k, carry):
        pltpu.sync_copy(etf.at[pl.ds(e0 + k * _STRIP, _STRIP)], tmp_v)

        def cbody(i, carry2):
            off = i * 16
            cidx_v[pl.ds(k * _STRIP + off, 16)] = (
                tmp_v[pl.ds(off, 16)] * _N
                + cidx_v[pl.ds(k * _STRIP + off, 16)]
                + (c2 * 2) * (_NR * _N))
            return carry2
        return lax.fori_loop(0, _STRIP // 16, cbody, carry)
    lax.fori_loop(0, _ET // _STRIP, sbody, 0)

    for l in range(2):
        c = c2 * 2 + l
        pltpu.sync_copy(s4.at[c, pl.ds(r0, _RPT)],
                        agg_sh.at[pl.ds(r0, _RPT)])

        @pl.when(s == _NS - 1)
        def _():
            pltpu.sync_copy(s4.at[c, pl.ds(_NS * _RPT, tail)],
                            agg_sh.at[pl.ds(_NS * _RPT, tail)])

        if l > 0:
            def abody(i, carry):
                off = i * 16
                cidx_v[pl.ds(off, 16)] = cidx_v[pl.ds(off, 16)] + _NR * _N
                return carry
            lax.fori_loop(0, _ET // 16, abody, 0)

        plsc.subcore_barrier()

        def fire(b, buf, sem):
            pltpu.async_copy(ylin.at[cidx_v.at[pl.ds(b * _BQ, _BQ)]],
                             buf, sem)

        def drain(buf, sem):
            pltpu.make_async_copy(ylin.at[cidx_v.at[pl.ds(0, _BQ)]],
                                  buf, sem).wait()

        fire(0, rows_a, sem_a)

        def bbody(j, carry):
            b0 = j * 2
            drain(rows_a, sem_a)
            fire(b0 + 1, rows_b, sem_b)
            pltpu.sync_copy(rows_a, agg_sh.at[dst_v.at[b0]], add=True)

            @pl.when(j < _NBQ // 2 - 1)
            def _():
                fire(b0 + 2, rows_a, sem_a)

            drain(rows_b, sem_b)
            pltpu.sync_copy(rows_b, agg_sh.at[dst_v.at[b0 + 1]], add=True)
            return carry
        lax.fori_loop(0, _NBQ // 2, bbody, 0)

        plsc.subcore_barrier()

        pltpu.sync_copy(agg_sh.at[pl.ds(r0, _RPT)],
                        out4.at[c, pl.ds(r0, _RPT)])

        @pl.when(s == _NS - 1)
        def _():
            pltpu.sync_copy(agg_sh.at[pl.ds(_NS * _RPT, tail)],
                            out4.at[c, pl.ds(_NS * _RPT, tail)])


def _sc_aggregate(ylin, s4, srcp, etp, dst2d):
    mesh = plsc.VectorSubcoreMesh(core_axis_name="c", subcore_axis_name="s")
    run = functools.partial(
        pl.kernel,
        out_type=jax.ShapeDtypeStruct((_NBK, _N, 128), jnp.float32),
        mesh=mesh,
        scratch_types=[
            pltpu.VMEM((_ET,), jnp.int32),
            pltpu.VMEM((_STRIP,), jnp.int32),
            pltpu.VMEM((_NBQ, _BQ), jnp.int32),
            pltpu.VMEM((_BQ, 128), jnp.float32),
            pltpu.VMEM((_BQ, 128), jnp.float32),
            pltpu.VMEM_SHARED((_AGG_ROWS, 128), jnp.float32),
            pltpu.SemaphoreType.DMA,
            pltpu.SemaphoreType.DMA,
        ],
    )(_sc_body)
    return run(ylin, s4, srcp, etp, dst2d)


def kernel(init_emb, W_bdd, W_loop, bias, edge_index, etype):
    f32 = jnp.float32
    xpad = (jnp.zeros((_N, _NBK, 128), f32)
            .at[:, :, :_BI].set(init_emb.reshape(_N, _NBK, _BI))
            .reshape(_N, _NBK * 128))
    wpad = jnp.zeros((_NR, _NBK, 128, _BO), f32).at[:, :, :_BI, :].set(W_bdd)
    bias8 = jnp.broadcast_to(bias.reshape(1, 512), (8, 512))

    src = edge_index[0]
    dst = edge_index[1]
    pad = _EP - _E
    srcp = jnp.concatenate([src, jnp.zeros((pad,), jnp.int32)])
    etp = jnp.concatenate([etype, jnp.zeros((pad,), jnp.int32)])
    dst2d = jnp.concatenate(
        [dst, jnp.full((pad,), _N, jnp.int32)]).reshape(_EP // _BQ, _BQ)

    y4 = _tc_messages(xpad, wpad)
    s4 = _tc_self(init_emb, W_loop, bias8)
    out4 = _sc_aggregate(y4.reshape(_NBK * _NR * _N, 128),
                         s4, srcp, etp, dst2d)
    return out4.transpose(1, 0, 2).reshape(_N, _NR * 128)


# speedup vs baseline: 2.7944x; 1.0522x over previous
import functools

import jax
import jax.numpy as jnp
from jax import lax
from jax.experimental import pallas as pl
from jax.experimental.pallas import tpu as pltpu
from jax.experimental.pallas import tpu_sc as plsc

_N = 10000
_E = 160000
_NR = 4
_NBK = 4
_BI = 75
_BO = 128
_EP = 163840
_NC = 2
_NS = 16
_ET = _EP // _NS
_BQ = 64
_NBQ = _ET // _BQ
_STRIP = 2048
_RPT = 624
_AGG_ROWS = _N + 8


def _msg_body(x_ref, w_ref, o_ref):
    o_ref[0, 0] = jnp.dot(x_ref[...], w_ref[0, 0],
                          preferred_element_type=jnp.float32)


def _tc_messages(xpad, wpad):
    tn = 400
    grid = (_N // tn, _NBK, _NR)
    return pl.pallas_call(
        _msg_body,
        grid=grid,
        in_specs=[
            pl.BlockSpec((tn, 128), lambda t, c, r: (t, c)),
            pl.BlockSpec((1, 1, 128, 128), lambda t, c, r: (r, c, 0, 0)),
        ],
        out_specs=pl.BlockSpec((1, 1, tn, 128), lambda t, c, r: (c, r, t, 0)),
        out_shape=jax.ShapeDtypeStruct((_NBK, _NR, _N, 128), jnp.float32),
    )(xpad, wpad)


def _self_body(x_ref, w_ref, b_ref, o_ref):
    o_ref[0] = jnp.dot(x_ref[...], w_ref[...],
                       preferred_element_type=jnp.float32) + b_ref[0:1, :]


def _tc_self(init_emb, w_loop, bias8):
    tn = 400
    grid = (_NBK, _N // tn)
    return pl.pallas_call(
        _self_body,
        grid=grid,
        in_specs=[
            pl.BlockSpec((tn, 300), lambda c, t: (t, 0)),
            pl.BlockSpec((300, 128), lambda c, t: (0, c)),
            pl.BlockSpec((8, 128), lambda c, t: (0, c)),
        ],
        out_specs=pl.BlockSpec((1, tn, 128), lambda c, t: (c, t, 0)),
        out_shape=jax.ShapeDtypeStruct((_NBK, _N, 128), jnp.float32),
    )(init_emb, w_loop, bias8)


def _sc_body(ylin, s4, srcf, etf, dst2d, out2,
             cidx_v, tmp_v, dst_v, rows_a, rows_b, agg_sh, sem_a, sem_b):
    c2 = lax.axis_index("c")
    s = lax.axis_index("s")
    e0 = s * _ET
    pltpu.sync_copy(srcf.at[pl.ds(e0, _ET)], cidx_v)
    pltpu.sync_copy(dst2d.at[pl.ds(s * _NBQ, _NBQ)], dst_v)
    r0 = s * _RPT
    tail = _N - _NS * _RPT

    def sbody(k, carry):
        pltpu.sync_copy(etf.at[pl.ds(e0 + k * _STRIP, _STRIP)], tmp_v)

        def cbody(i, carry2):
            off = i * 16
            cidx_v[pl.ds(k * _STRIP + off, 16)] = (
                tmp_v[pl.ds(off, 16)] * _N
                + cidx_v[pl.ds(k * _STRIP + off, 16)]
                + (c2 * 2) * (_NR * _N))
            return carry2
        return lax.fori_loop(0, _STRIP // 16, cbody, carry)
    lax.fori_loop(0, _ET // _STRIP, sbody, 0)

    for l in range(2):
        c = c2 * 2 + l
        pltpu.sync_copy(s4.at[c, pl.ds(r0, _RPT)],
                        agg_sh.at[pl.ds(r0, _RPT)])

        @pl.when(s == _NS - 1)
        def _():
            pltpu.sync_copy(s4.at[c, pl.ds(_NS * _RPT, tail)],
                            agg_sh.at[pl.ds(_NS * _RPT, tail)])

        if l > 0:
            def abody(i, carry):
                off = i * 16
                cidx_v[pl.ds(off, 16)] = cidx_v[pl.ds(off, 16)] + _NR * _N
                return carry
            lax.fori_loop(0, _ET // 16, abody, 0)

        plsc.subcore_barrier()

        def fire(b, buf, sem):
            pltpu.async_copy(ylin.at[cidx_v.at[pl.ds(b * _BQ, _BQ)]],
                             buf, sem)

        def drain(buf, sem):
            pltpu.make_async_copy(ylin.at[cidx_v.at[pl.ds(0, _BQ)]],
                                  buf, sem).wait()

        fire(0, rows_a, sem_a)

        def bbody(j, carry):
            b0 = j * 2
            drain(rows_a, sem_a)
            fire(b0 + 1, rows_b, sem_b)
            pltpu.sync_copy(rows_a, agg_sh.at[dst_v.at[b0]], add=True)

            @pl.when(j < _NBQ // 2 - 1)
            def _():
                fire(b0 + 2, rows_a, sem_a)

            drain(rows_b, sem_b)
            pltpu.sync_copy(rows_b, agg_sh.at[dst_v.at[b0 + 1]], add=True)
            return carry
        lax.fori_loop(0, _NBQ // 2, bbody, 0)

        plsc.subcore_barrier()

        pltpu.sync_copy(agg_sh.at[pl.ds(r0, _RPT)],
                        out2.at[pl.ds(r0, _RPT), pl.ds(c * 128, 128)])

        @pl.when(s == _NS - 1)
        def _():
            pltpu.sync_copy(
                agg_sh.at[pl.ds(_NS * _RPT, tail)],
                out2.at[pl.ds(_NS * _RPT, tail), pl.ds(c * 128, 128)])


def _sc_aggregate(ylin, s4, srcp, etp, dst2d):
    mesh = plsc.VectorSubcoreMesh(core_axis_name="c", subcore_axis_name="s")
    run = functools.partial(
        pl.kernel,
        out_type=jax.ShapeDtypeStruct((_N, _NR * 128), jnp.float32),
        mesh=mesh,
        scratch_types=[
            pltpu.VMEM((_ET,), jnp.int32),
            pltpu.VMEM((_STRIP,), jnp.int32),
            pltpu.VMEM((_NBQ, _BQ), jnp.int32),
            pltpu.VMEM((_BQ, 128), jnp.float32),
            pltpu.VMEM((_BQ, 128), jnp.float32),
            pltpu.VMEM_SHARED((_AGG_ROWS, 128), jnp.float32),
            pltpu.SemaphoreType.DMA,
            pltpu.SemaphoreType.DMA,
        ],
    )(_sc_body)
    return run(ylin, s4, srcp, etp, dst2d)


def kernel(init_emb, W_bdd, W_loop, bias, edge_index, etype):
    f32 = jnp.float32
    xpad = (jnp.zeros((_N, _NBK, 128), f32)
            .at[:, :, :_BI].set(init_emb.reshape(_N, _NBK, _BI))
            .reshape(_N, _NBK * 128))
    wpad = jnp.zeros((_NR, _NBK, 128, _BO), f32).at[:, :, :_BI, :].set(W_bdd)
    bias8 = jnp.broadcast_to(bias.reshape(1, 512), (8, 512))

    src = edge_index[0]
    dst = edge_index[1]
    pad = _EP - _E
    srcp = jnp.concatenate([src, jnp.zeros((pad,), jnp.int32)])
    etp = jnp.concatenate([etype, jnp.zeros((pad,), jnp.int32)])
    dst2d = jnp.concatenate(
        [dst, jnp.full((pad,), _N, jnp.int32)]).reshape(_EP // _BQ, _BQ)

    y4 = _tc_messages(xpad, wpad)
    s4 = _tc_self(init_emb, W_loop, bias8)
    return _sc_aggregate(y4.reshape(_NBK * _NR * _N, 128),
                         s4, srcp, etp, dst2d)


# speedup vs baseline: 3.7208x; 1.3315x over previous
import functools

import jax
import jax.numpy as jnp
from jax import lax
from jax.experimental import pallas as pl
from jax.experimental.pallas import tpu as pltpu
from jax.experimental.pallas import tpu_sc as plsc

_N = 10000
_E = 160000
_NR = 4
_NBK = 4
_BI = 75
_BO = 128
_EP = 163840
_NC = 2
_NS = 16
_ET = _EP // _NS
_BQ = 64
_NBQ = _ET // _BQ
_STRIP = 2048
_RPT = 624
_AGG_ROWS = _N + 8


def _tc_body(x_ref, wb_ref, wl_ref, b_ref, y_ref, s_ref):
    r = pl.program_id(1)

    @pl.when(r < _NR)
    def _():
        for c in range(_NBK):
            y_ref[c, 0] = jnp.dot(x_ref[:, c * _BI:(c + 1) * _BI],
                                  wb_ref[0, c],
                                  preferred_element_type=jnp.float32)

    @pl.when(r == _NR)
    def _():
        for c in range(_NBK):
            s_ref[c] = (jnp.dot(x_ref[...], wl_ref[:, c * 128:(c + 1) * 128],
                                preferred_element_type=jnp.float32)
                        + b_ref[0:1, c * 128:(c + 1) * 128])


def _tc_stage(init_emb, w_bdd, w_loop, bias8):
    tn = 400
    grid = (_N // tn, _NR + 1)
    return pl.pallas_call(
        _tc_body,
        grid=grid,
        in_specs=[
            pl.BlockSpec((tn, 300), lambda t, r: (t, 0)),
            pl.BlockSpec((1, _NBK, _BI, _BO),
                         lambda t, r: (jnp.minimum(r, _NR - 1), 0, 0, 0)),
            pl.BlockSpec((300, 512), lambda t, r: (0, 0)),
            pl.BlockSpec((8, 512), lambda t, r: (0, 0)),
        ],
        out_specs=[
            pl.BlockSpec((_NBK, 1, tn, 128),
                         lambda t, r: (0, jnp.minimum(r, _NR - 1), t, 0)),
            pl.BlockSpec((_NBK, tn, 128), lambda t, r: (0, t, 0)),
        ],
        out_shape=[
            jax.ShapeDtypeStruct((_NBK, _NR, _N, 128), jnp.float32),
            jax.ShapeDtypeStruct((_NBK, _N, 128), jnp.float32),
        ],
    )(init_emb, w_bdd, w_loop, bias8)


def _sc_body(ylin, s4, srcf, etf, dst2d, out2,
             cidx_v, tmp_v, dst_v, rows_a, rows_b, agg_sh, sem_a, sem_b):
    c2 = lax.axis_index("c")
    s = lax.axis_index("s")
    e0 = s * _ET
    pltpu.sync_copy(srcf.at[pl.ds(e0, _ET)], cidx_v)
    pltpu.sync_copy(dst2d.at[pl.ds(s * _NBQ, _NBQ)], dst_v)
    r0 = s * _RPT
    tail = _N - _NS * _RPT

    def sbody(k, carry):
        pltpu.sync_copy(etf.at[pl.ds(e0 + k * _STRIP, _STRIP)], tmp_v)

        def cbody(i, carry2):
            off = i * 16
            cidx_v[pl.ds(k * _STRIP + off, 16)] = (
                tmp_v[pl.ds(off, 16)] * _N
                + cidx_v[pl.ds(k * _STRIP + off, 16)]
                + (c2 * 2) * (_NR * _N))
            return carry2
        return lax.fori_loop(0, _STRIP // 16, cbody, carry)
    lax.fori_loop(0, _ET // _STRIP, sbody, 0)

    for l in range(2):
        c = c2 * 2 + l
        pltpu.sync_copy(s4.at[c, pl.ds(r0, _RPT)],
                        agg_sh.at[pl.ds(r0, _RPT)])

        @pl.when(s == _NS - 1)
        def _():
            pltpu.sync_copy(s4.at[c, pl.ds(_NS * _RPT, tail)],
                            agg_sh.at[pl.ds(_NS * _RPT, tail)])

        if l > 0:
            def abody(i, carry):
                off = i * 16
                cidx_v[pl.ds(off, 16)] = cidx_v[pl.ds(off, 16)] + _NR * _N
                return carry
            lax.fori_loop(0, _ET // 16, abody, 0)

        plsc.subcore_barrier()

        def fire(b, buf, sem):
            pltpu.async_copy(ylin.at[cidx_v.at[pl.ds(b * _BQ, _BQ)]],
                             buf, sem)

        def drain(buf, sem):
            pltpu.make_async_copy(ylin.at[cidx_v.at[pl.ds(0, _BQ)]],
                                  buf, sem).wait()

        fire(0, rows_a, sem_a)

        def bbody(j, carry):
            b0 = j * 2
            drain(rows_a, sem_a)
            fire(b0 + 1, rows_b, sem_b)
            pltpu.sync_copy(rows_a, agg_sh.at[dst_v.at[b0]], add=True)

            @pl.when(j < _NBQ // 2 - 1)
            def _():
                fire(b0 + 2, rows_a, sem_a)

            drain(rows_b, sem_b)
            pltpu.sync_copy(rows_b, agg_sh.at[dst_v.at[b0 + 1]], add=True)
            return carry
        lax.fori_loop(0, _NBQ // 2, bbody, 0)

        plsc.subcore_barrier()

        pltpu.sync_copy(agg_sh.at[pl.ds(r0, _RPT)],
                        out2.at[pl.ds(r0, _RPT), pl.ds(c * 128, 128)])

        @pl.when(s == _NS - 1)
        def _():
            pltpu.sync_copy(
                agg_sh.at[pl.ds(_NS * _RPT, tail)],
                out2.at[pl.ds(_NS * _RPT, tail), pl.ds(c * 128, 128)])


def _sc_aggregate(ylin, s4, srcp, etp, dst2d):
    mesh = plsc.VectorSubcoreMesh(core_axis_name="c", subcore_axis_name="s")
    run = functools.partial(
        pl.kernel,
        out_type=jax.ShapeDtypeStruct((_N, _NR * 128), jnp.float32),
        mesh=mesh,
        scratch_types=[
            pltpu.VMEM((_ET,), jnp.int32),
            pltpu.VMEM((_STRIP,), jnp.int32),
            pltpu.VMEM((_NBQ, _BQ), jnp.int32),
            pltpu.VMEM((_BQ, 128), jnp.float32),
            pltpu.VMEM((_BQ, 128), jnp.float32),
            pltpu.VMEM_SHARED((_AGG_ROWS, 128), jnp.float32),
            pltpu.SemaphoreType.DMA,
            pltpu.SemaphoreType.DMA,
        ],
    )(_sc_body)
    return run(ylin, s4, srcp, etp, dst2d)


def kernel(init_emb, W_bdd, W_loop, bias, edge_index, etype):
    bias8 = jnp.broadcast_to(bias.reshape(1, 512), (8, 512))

    src = edge_index[0]
    dst = edge_index[1]
    pad = _EP - _E
    srcp = jnp.concatenate([src, jnp.zeros((pad,), jnp.int32)])
    etp = jnp.concatenate([etype, jnp.zeros((pad,), jnp.int32)])
    dst2d = jnp.concatenate(
        [dst, jnp.full((pad,), _N, jnp.int32)]).reshape(_EP // _BQ, _BQ)

    y4, s4 = _tc_stage(init_emb, W_bdd, W_loop, bias8)
    return _sc_aggregate(y4.reshape(_NBK * _NR * _N, 128),
                         s4, srcp, etp, dst2d)
